# Initial kernel scaffold; baseline (speedup 1.0000x reference)
import jax
import jax.numpy as jnp
from jax.experimental import pallas as pl


def _rowmax_body(x_ref, o_ref):
    x = x_ref[0]
    o_ref[0, :] = jnp.max(x[:, :91], axis=1)


def kernel(pred_logits, pred_boxes, target_sizes):
    bs, n, c = pred_logits.shape
    cat = c - 1
    rows_per_block = 1250
    rowmax = pl.pallas_call(
        _rowmax_body,
        grid=(bs, n // rows_per_block),
        in_specs=[pl.BlockSpec((1, rows_per_block, c), lambda b, r: (b, r, 0))],
        out_specs=pl.BlockSpec((1, rows_per_block), lambda b, r: (b, r)),
        out_shape=jax.ShapeDtypeStruct((bs, n), jnp.float32),
    )(pred_logits)

    _, rowidx = jax.lax.top_k(rowmax, 300)
    gathered = jnp.take_along_axis(
        pred_logits[..., :cat], rowidx[:, :, None], axis=1
    )
    vals, flat = jax.lax.top_k(gathered.reshape(bs, -1), 300)
    scores = jax.nn.sigmoid(vals)
    labels = flat % cat
    box_rows = jnp.take_along_axis(rowidx, flat // cat, axis=1)

    b = jnp.take_along_axis(
        pred_boxes, box_rows[:, :, None], axis=1
    )
    cx, cy, w, h = b[..., 0], b[..., 1], b[..., 2], b[..., 3]
    xyxy = jnp.stack(
        [cx - 0.5 * w, cy - 0.5 * h, cx + 0.5 * w, cy + 0.5 * h], axis=-1
    )
    img_h = target_sizes[:, 0]
    img_w = target_sizes[:, 1]
    scale = jnp.stack([img_w, img_h, img_w, img_h], axis=1)
    boxes = xyxy * scale[:, None, :]
    return scores, labels, boxes


# speedup vs baseline: 17.6728x; 17.6728x over previous
import jax
import jax.numpy as jnp
from jax.experimental import pallas as pl


def _rowmax_body(x_ref, o_ref):
    x = x_ref[0]
    o_ref[0, 0, :] = jnp.max(x[:, :91], axis=1)


def kernel(pred_logits, pred_boxes, target_sizes):
    bs, n, c = pred_logits.shape
    cat = c - 1
    rows_per_block = 1000
    nblk = n // rows_per_block
    rowmax = pl.pallas_call(
        _rowmax_body,
        grid=(bs, nblk),
        in_specs=[pl.BlockSpec((1, rows_per_block, c), lambda b, r: (b, r, 0))],
        out_specs=pl.BlockSpec(
            (1, 1, rows_per_block), lambda b, r: (b * nblk + r, 0, 0)
        ),
        out_shape=jax.ShapeDtypeStruct((bs * nblk, 1, rows_per_block), jnp.float32),
    )(pred_logits)
    rowmax = rowmax.reshape(bs, n)

    _, rowidx = jax.lax.top_k(rowmax, 300)
    gathered = jnp.take_along_axis(
        pred_logits[..., :cat], rowidx[:, :, None], axis=1
    )
    vals, flat = jax.lax.top_k(gathered.reshape(bs, -1), 300)
    scores = jax.nn.sigmoid(vals)
    labels = flat % cat
    box_rows = jnp.take_along_axis(rowidx, flat // cat, axis=1)

    b = jnp.take_along_axis(
        pred_boxes, box_rows[:, :, None], axis=1
    )
    cx, cy, w, h = b[..., 0], b[..., 1], b[..., 2], b[..., 3]
    xyxy = jnp.stack(
        [cx - 0.5 * w, cy - 0.5 * h, cx + 0.5 * w, cy + 0.5 * h], axis=-1
    )
    img_h = target_sizes[:, 0]
    img_w = target_sizes[:, 1]
    scale = jnp.stack([img_w, img_h, img_w, img_h], axis=1)
    boxes = xyxy * scale[:, None, :]
    return scores, labels, boxes


# speedup vs baseline: 18.0283x; 1.0201x over previous
import math

import jax
import jax.numpy as jnp
from jax import lax
from jax.experimental import pallas as pl
from jax.experimental.pallas import tpu as pltpu
from jax.experimental.pallas import tpu_sc as plsc

N_ROWS = 20000
N_CLS = 91
ROWS_PER_BLK = 1000
GRP = 32
N_GRP = N_ROWS // GRP
GRP_VREGS = (N_GRP + 15) // 16
CAND_CAP = 512
DEEP_CAP = 64
OUT_W = 304
NEG_INF = float("-inf")


def _rowstats_body(x_ref, m_ref, a_ref, m2_ref):
    x = x_ref[0][:, :N_CLS]
    m = jnp.max(x, axis=1)
    col = lax.broadcasted_iota(jnp.int32, x.shape, 1)
    a = jnp.min(jnp.where(x == m[:, None], col, N_CLS), axis=1)
    x2 = jnp.where(col == a[:, None], NEG_INF, x)
    m2 = jnp.max(x2, axis=1)
    m_ref[0, 0, :] = m
    a_ref[0, 0, :] = a
    m2_ref[0, 0, :] = m2


def _key(x):
    b = lax.bitcast_convert_type(x, jnp.uint32)
    flip = jnp.where(x < 0.0, jnp.uint32(0xFFFFFFFF), jnp.uint32(0x80000000))
    return b ^ flip


def _unkey(u):
    flip = jnp.where(
        u >= jnp.uint32(0x80000000), jnp.uint32(0x80000000), jnp.uint32(0xFFFFFFFF)
    )
    return lax.bitcast_convert_type(u ^ flip, jnp.float32)


def _shuf(x, idx):
    dn = lax.GatherDimensionNumbers(
        offset_dims=(), collapsed_slice_dims=(0,), start_index_map=(0,)
    )
    return lax.gather(
        x,
        idx[:, None],
        dimension_numbers=dn,
        slice_sizes=(1,),
        mode=lax.GatherScatterMode.PROMISE_IN_BOUNDS,
    )


def _scalar(v):
    return jnp.max(v)


def _popcount(m):
    return _scalar(plsc.all_reduce_population_count(m))


def _sc_body(rm_hbm, am_hbm, m2_hbm, logits_hbm, boxes_hbm, ts_hbm,
             scores_out, labels_out, boxes_out,
             rm_v, am_v, m2_v, gmax_v, sortk_v, sortv_v,
             deepr_v, deepbuf_v, boxidx_v, bidx_v, boxrows_v,
             scores_v, labels_v, boxout_v, ts_v, sem):
    nc = 2
    wid = lax.axis_index("s") * nc + lax.axis_index("c")
    lane = lax.iota(jnp.int32, 16)

    @pl.when(wid < 16)
    def _work():
        b = wid
        pltpu.sync_copy(rm_hbm.at[b], rm_v)
        pltpu.sync_copy(am_hbm.at[b], am_v)
        pltpu.sync_copy(m2_hbm.at[b], m2_v)
        pltpu.sync_copy(ts_hbm.at[b], ts_v)

        def grp_body(j, _):
            gid = j * 16 + lane
            valid = gid < N_GRP

            def inner(k, acc):
                g = plsc.load_gather(
                    rm_v, [jnp.minimum(gid * GRP + k, N_ROWS - 1)]
                )
                return jnp.maximum(acc, g)

            acc = lax.fori_loop(0, GRP, inner, jnp.full((16,), NEG_INF, jnp.float32))
            gk = jnp.where(valid, _key(acc), jnp.uint32(0))
            gmax_v[pl.ds(j * 16, 16)] = gk
            return 0

        lax.fori_loop(0, GRP_VREGS, grp_body, 0)

        def count_ge(T):
            def cbody(i, acc):
                k = gmax_v[pl.ds(i * 16, 16)]
                return acc + jnp.where(k >= T, 1, 0).astype(jnp.int32)

            accv = lax.fori_loop(0, GRP_VREGS, cbody, jnp.zeros((16,), jnp.int32))
            return jnp.sum(accv)

        def bis_body(i, T):
            bit = 31 - i
            cand = T | (jnp.uint32(1) << bit.astype(jnp.uint32))
            c = count_ge(cand)
            return jnp.where(c >= 300, cand, T)

        t2 = lax.fori_loop(0, 24, bis_body, jnp.uint32(0))

        def zero_body(j, _):
            sortk_v[pl.ds(j * 16, 16)] = jnp.zeros((16,), jnp.uint32)
            sortv_v[pl.ds(j * 16, 16)] = jnp.zeros((16,), jnp.int32)
            return 0

        lax.fori_loop(0, CAND_CAP // 16, zero_body, 0)

        def dinit_body(j, _):
            deepr_v[pl.ds(j * 16, 16)] = j * 16 + lane
            return 0

        lax.fori_loop(0, DEEP_CAP // 16, dinit_body, 0)

        def cmp_body(i, carry):
            off, offd = carry
            x = rm_v[pl.ds(i * 16, 16)]
            u = _key(x)
            m = u >= t2

            def do_store(carry):
                off, offd = carry
                am = am_v[pl.ds(i * 16, 16)]
                flat = (i * 16 + lane) * N_CLS + am
                offc = jnp.minimum(off, CAND_CAP - 16)
                plsc.store_compressed(sortk_v.at[pl.ds(offc, 16)], u, mask=m)
                plsc.store_compressed(sortv_v.at[pl.ds(offc, 16)], flat, mask=m)
                m2u = _key(m2_v[pl.ds(i * 16, 16)])
                md = m & (m2u >= t2)
                offdc = jnp.minimum(offd, DEEP_CAP - 16)
                plsc.store_compressed(
                    deepr_v.at[pl.ds(offdc, 16)], i * 16 + lane, mask=md
                )
                return off + _popcount(m), offd + _popcount(md)

            return lax.cond(jnp.any(m), do_store, lambda c: c, (off, offd))

        n_cand, n_deep = lax.fori_loop(
            0, N_ROWS // 16, cmp_body, (jnp.int32(0), jnp.int32(0))
        )
        n_deep = jnp.minimum(n_deep, DEEP_CAP)

        def deep_row(dr, off):
            zero16 = jnp.zeros((16,), jnp.int32)
            r_vec = plsc.load_gather(deepr_v, [zero16 + dr])
            am_vec = plsc.load_gather(am_v, [r_vec])
            flat_base = r_vec * N_CLS
            r_s = jnp.max(r_vec)
            start = (b * N_ROWS + r_s) * 92
            al = pl.multiple_of(start & ~jnp.int32(7), 8)
            delta = start - al
            pltpu.sync_copy(logits_hbm.at[pl.ds(al, 104)], deepbuf_v)

            def deep_chunk(ci, off):
                cls = ci * 16 + lane
                ok = cls < N_CLS
                v = plsc.load_gather(deepbuf_v, [delta + cls])
                u = _key(v)
                m = ok & (cls != am_vec) & (u >= t2)

                def dstore(off):
                    offc = jnp.minimum(off, CAND_CAP - 16)
                    plsc.store_compressed(sortk_v.at[pl.ds(offc, 16)], u, mask=m)
                    plsc.store_compressed(
                        sortv_v.at[pl.ds(offc, 16)], flat_base + cls, mask=m
                    )
                    return off + _popcount(m)

                return lax.cond(jnp.any(m), dstore, lambda o: o, off)

            return lax.fori_loop(0, 6, deep_chunk, off)

        n_cand = lax.fori_loop(0, n_deep, deep_row, n_cand)

        nv = CAND_CAP // 16

        def inter_stage(ksz, j):
            jb = j // 16
            s = int(math.log2(jb)) if jb > 0 else 0

            def pair_body(t, _):
                v = ((t >> s) << (s + 1)) | (t & (jb - 1))
                p = v | jb
                ka = sortk_v[pl.ds(v * 16, 16)]
                va = sortv_v[pl.ds(v * 16, 16)]
                kb = sortk_v[pl.ds(p * 16, 16)]
                vb = sortv_v[pl.ds(p * 16, 16)]
                dir_asc = ((v * 16) & ksz) == 0
                lo_before = (ka > kb) | ((ka == kb) & (va < vb))
                swap = lo_before ^ dir_asc
                sortk_v[pl.ds(v * 16, 16)] = jnp.where(swap, kb, ka)
                sortv_v[pl.ds(v * 16, 16)] = jnp.where(swap, vb, va)
                sortk_v[pl.ds(p * 16, 16)] = jnp.where(swap, ka, kb)
                sortv_v[pl.ds(p * 16, 16)] = jnp.where(swap, va, vb)
                return 0

            lax.fori_loop(0, nv // 2, pair_body, 0)

        def intra_stage(ksz, j):
            pidx = lane ^ j

            def vreg_body(v, _):
                ka = sortk_v[pl.ds(v * 16, 16)]
                va = sortv_v[pl.ds(v * 16, 16)]
                kb = _shuf(ka, pidx)
                vb = _shuf(va, pidx)
                am_lower = (lane & j) == 0
                klo = jnp.where(am_lower, ka, kb)
                khi = jnp.where(am_lower, kb, ka)
                vlo = jnp.where(am_lower, va, vb)
                vhi = jnp.where(am_lower, vb, va)
                dir_asc = (((v * 16 + lane) & ksz) == 0)
                lo_before = (klo > khi) | ((klo == khi) & (vlo < vhi))
                swap = lo_before ^ dir_asc
                sortk_v[pl.ds(v * 16, 16)] = jnp.where(swap, kb, ka)
                sortv_v[pl.ds(v * 16, 16)] = jnp.where(swap, vb, va)
                return 0

            lax.fori_loop(0, nv, vreg_body, 0)

        ksz = 2
        while ksz <= CAND_CAP:
            j = ksz // 2
            while j >= 1:
                if j >= 16:
                    inter_stage(ksz, j)
                else:
                    intra_stage(ksz, j)
                j //= 2
            ksz *= 2

        inv91 = jnp.float32(1.0 / N_CLS)

        def out_body(jv, _):
            u = sortk_v[pl.ds(jv * 16, 16)]
            fl = sortv_v[pl.ds(jv * 16, 16)]
            x = _unkey(u)
            scores_v[pl.ds(jv * 16, 16)] = 1.0 / (1.0 + jnp.exp(-x))
            br = (fl.astype(jnp.float32) * inv91).astype(jnp.int32)
            labels_v[pl.ds(jv * 16, 16)] = fl - br * N_CLS
            boxidx_v[pl.ds(jv * 16, 16)] = (b * N_ROWS + br) * 4
            return 0

        lax.fori_loop(0, OUT_W // 16, out_body, 0)

        pltpu.sync_copy(scores_v, scores_out.at[b])
        pltpu.sync_copy(labels_v, labels_out.at[b])

        def bidx_body(jv, _):
            pos = jv * 16 + lane
            base = plsc.load_gather(boxidx_v, [pos >> 2])
            bidx_v[pl.ds(jv * 16, 16)] = base + (pos & 3)
            return 0

        lax.fori_loop(0, OUT_W * 4 // 16, bidx_body, 0)
        pltpu.async_copy(boxes_hbm.at[bidx_v], boxrows_v, sem).wait()

        sc_vec = _shuf(ts_v[pl.ds(0, 16)], (lane & 1) ^ 1)

        def box_body(jv, _):
            pos = jv * 16 + lane
            cl = pos & 3
            v = boxrows_v[pl.ds(jv * 16, 16)]
            vp = plsc.load_gather(boxrows_v, [pos ^ 2])
            xy = jnp.where(cl < 2, v - 0.5 * vp, vp + 0.5 * v)
            boxout_v[pl.ds(jv * 16, 16)] = xy * sc_vec
            return 0

        lax.fori_loop(0, OUT_W * 4 // 16, box_body, 0)
        pltpu.sync_copy(boxout_v, boxes_out.at[b])


def _run_sc(rm, am, m2, logits_flat, boxes_flat, ts_pad):
    mesh = plsc.VectorSubcoreMesh(core_axis_name="c", subcore_axis_name="s")
    f = pl.kernel(
        _sc_body,
        mesh=mesh,
        compiler_params=pltpu.CompilerParams(needs_layout_passes=False),
        out_type=[
            jax.ShapeDtypeStruct((16, OUT_W), jnp.float32),
            jax.ShapeDtypeStruct((16, OUT_W), jnp.int32),
            jax.ShapeDtypeStruct((16, OUT_W * 4), jnp.float32),
        ],
        scratch_types=[
            pltpu.VMEM((N_ROWS,), jnp.float32),
            pltpu.VMEM((N_ROWS,), jnp.int32),
            pltpu.VMEM((N_ROWS,), jnp.float32),
            pltpu.VMEM((GRP_VREGS * 16,), jnp.uint32),
            pltpu.VMEM((CAND_CAP,), jnp.uint32),
            pltpu.VMEM((CAND_CAP,), jnp.int32),
            pltpu.VMEM((DEEP_CAP,), jnp.int32),
            pltpu.VMEM((104,), jnp.float32),
            pltpu.VMEM((OUT_W,), jnp.int32),
            pltpu.VMEM((OUT_W * 4,), jnp.int32),
            pltpu.VMEM((OUT_W * 4,), jnp.float32),
            pltpu.VMEM((OUT_W,), jnp.float32),
            pltpu.VMEM((OUT_W,), jnp.int32),
            pltpu.VMEM((OUT_W * 4,), jnp.float32),
            pltpu.VMEM((16,), jnp.float32),
            pltpu.SemaphoreType.DMA,
        ],
    )
    return f(rm, am, m2, logits_flat, boxes_flat, ts_pad)


def kernel(pred_logits, pred_boxes, target_sizes):
    bs, n, c = pred_logits.shape
    nblk = n // ROWS_PER_BLK
    out3 = jax.ShapeDtypeStruct((bs * nblk, 1, ROWS_PER_BLK), jnp.float32)
    out3i = jax.ShapeDtypeStruct((bs * nblk, 1, ROWS_PER_BLK), jnp.int32)
    rm, am, m2 = pl.pallas_call(
        _rowstats_body,
        grid=(bs, nblk),
        in_specs=[pl.BlockSpec((1, ROWS_PER_BLK, c), lambda b, r: (b, r, 0))],
        out_specs=[
            pl.BlockSpec((1, 1, ROWS_PER_BLK), lambda b, r: (b * nblk + r, 0, 0))
        ] * 3,
        out_shape=[out3, out3i, out3],
    )(pred_logits)
    rm = rm.reshape(bs, n)
    am = am.reshape(bs, n)
    m2 = m2.reshape(bs, n)

    logits_flat = pred_logits.reshape(bs * n * c)
    boxes_flat = pred_boxes.reshape(bs * n * 4)
    ts_pad = jnp.pad(target_sizes, ((0, 0), (0, 14)))

    scores_p, labels_p, boxes_p = _run_sc(
        rm, am, m2, logits_flat, boxes_flat, ts_pad
    )
    scores = scores_p[:, :300]
    labels = labels_p[:, :300]
    boxes = boxes_p.reshape(bs, OUT_W, 4)[:, :300]
    return scores, labels, boxes


# speedup vs baseline: 22.6088x; 1.2541x over previous
import math

import jax
import jax.numpy as jnp
from jax import lax
from jax.experimental import pallas as pl
from jax.experimental.pallas import tpu as pltpu
from jax.experimental.pallas import tpu_sc as plsc

N_ROWS = 20000
N_CLS = 91
ROWS_PER_BLK = 1000
GRP = 32
N_GRP = N_ROWS // GRP
GRP_VREGS = (N_GRP + 15) // 16
CAND_CAP = 512
DEEP_CAP = 64
OUT_W = 304
NEG_INF = float("-inf")


def _rowstats_body(x_ref, m_ref, a_ref, m2_ref):
    x = x_ref[0, :N_CLS]
    m = jnp.max(x, axis=0)
    row = lax.broadcasted_iota(jnp.int32, x.shape, 0)
    a = jnp.min(jnp.where(x == m[None, :], row, N_CLS), axis=0)
    x2 = jnp.where(row == a[None, :], NEG_INF, x)
    m2 = jnp.max(x2, axis=0)
    m_ref[0, 0, :] = m
    a_ref[0, 0, :] = a
    m2_ref[0, 0, :] = m2


def _key(x):
    b = lax.bitcast_convert_type(x, jnp.uint32)
    flip = jnp.where(x < 0.0, jnp.uint32(0xFFFFFFFF), jnp.uint32(0x80000000))
    return b ^ flip


def _unkey(u):
    flip = jnp.where(
        u >= jnp.uint32(0x80000000), jnp.uint32(0x80000000), jnp.uint32(0xFFFFFFFF)
    )
    return lax.bitcast_convert_type(u ^ flip, jnp.float32)


def _shuf(x, idx):
    dn = lax.GatherDimensionNumbers(
        offset_dims=(), collapsed_slice_dims=(0,), start_index_map=(0,)
    )
    return lax.gather(
        x,
        idx[:, None],
        dimension_numbers=dn,
        slice_sizes=(1,),
        mode=lax.GatherScatterMode.PROMISE_IN_BOUNDS,
    )


def _scalar(v):
    return jnp.max(v)


def _popcount(m):
    return _scalar(plsc.all_reduce_population_count(m))


def _sc_body(rm_hbm, am_hbm, m2_hbm, logits_hbm, boxes_hbm, ts_hbm,
             scores_out, labels_out, boxes_out,
             rm_v, am_v, m2_v, gmax_v, sortk_v, sortv_v,
             deepr_v, deepbuf_v, boxidx_v, bidx_v, boxrows_v,
             scores_v, labels_v, boxout_v, ts_v, sem):
    nc = 2
    wid = lax.axis_index("s") * nc + lax.axis_index("c")
    lane = lax.iota(jnp.int32, 16)

    @pl.when(wid < 16)
    def _work():
        b = wid
        pltpu.sync_copy(rm_hbm.at[b], rm_v)
        pltpu.sync_copy(am_hbm.at[b], am_v)
        pltpu.sync_copy(m2_hbm.at[b], m2_v)
        pltpu.sync_copy(ts_hbm.at[b], ts_v)

        def grp_body(j, _):
            gid = j * 16 + lane
            valid = gid < N_GRP

            def inner(k, acc):
                g = plsc.load_gather(
                    rm_v, [jnp.minimum(gid * GRP + k, N_ROWS - 1)]
                )
                return jnp.maximum(acc, g)

            acc = lax.fori_loop(0, GRP, inner, jnp.full((16,), NEG_INF, jnp.float32))
            gk = jnp.where(valid, _key(acc), jnp.uint32(0))
            gmax_v[pl.ds(j * 16, 16)] = gk
            return 0

        lax.fori_loop(0, GRP_VREGS, grp_body, 0)

        def count_ge(T):
            def cbody(i, acc):
                k = gmax_v[pl.ds(i * 16, 16)]
                return acc + jnp.where(k >= T, 1, 0).astype(jnp.int32)

            accv = lax.fori_loop(0, GRP_VREGS, cbody, jnp.zeros((16,), jnp.int32))
            return jnp.sum(accv)

        def bis_body(i, T):
            bit = 31 - i
            cand = T | (jnp.uint32(1) << bit.astype(jnp.uint32))
            c = count_ge(cand)
            return jnp.where(c >= 300, cand, T)

        t2 = lax.fori_loop(0, 24, bis_body, jnp.uint32(0))

        def zero_body(j, _):
            sortk_v[pl.ds(j * 16, 16)] = jnp.zeros((16,), jnp.uint32)
            sortv_v[pl.ds(j * 16, 16)] = jnp.zeros((16,), jnp.int32)
            return 0

        lax.fori_loop(0, CAND_CAP // 16, zero_body, 0)

        def dinit_body(j, _):
            deepr_v[pl.ds(j * 16, 16)] = j * 16 + lane
            return 0

        lax.fori_loop(0, DEEP_CAP // 16, dinit_body, 0)

        def cmp_body(i, carry):
            off, offd = carry
            x = rm_v[pl.ds(i * 16, 16)]
            u = _key(x)
            m = u >= t2

            def do_store(carry):
                off, offd = carry
                am = am_v[pl.ds(i * 16, 16)]
                flat = (i * 16 + lane) * N_CLS + am
                offc = jnp.minimum(off, CAND_CAP - 16)
                plsc.store_compressed(sortk_v.at[pl.ds(offc, 16)], u, mask=m)
                plsc.store_compressed(sortv_v.at[pl.ds(offc, 16)], flat, mask=m)
                m2u = _key(m2_v[pl.ds(i * 16, 16)])
                md = m & (m2u >= t2)
                offdc = jnp.minimum(offd, DEEP_CAP - 16)
                plsc.store_compressed(
                    deepr_v.at[pl.ds(offdc, 16)], i * 16 + lane, mask=md
                )
                return off + _popcount(m), offd + _popcount(md)

            return lax.cond(jnp.any(m), do_store, lambda c: c, (off, offd))

        n_cand, n_deep = lax.fori_loop(
            0, N_ROWS // 16, cmp_body, (jnp.int32(0), jnp.int32(0))
        )
        n_deep = jnp.minimum(n_deep, DEEP_CAP)

        def deep_row(dr, off):
            zero16 = jnp.zeros((16,), jnp.int32)
            r_vec = plsc.load_gather(deepr_v, [zero16 + dr])
            am_vec = plsc.load_gather(am_v, [r_vec])
            flat_base = r_vec * N_CLS
            r_s = jnp.max(r_vec)
            start = (b * N_ROWS + r_s) * 92
            al = pl.multiple_of(start & ~jnp.int32(7), 8)
            delta = start - al
            pltpu.sync_copy(logits_hbm.at[pl.ds(al, 104)], deepbuf_v)

            def deep_chunk(ci, off):
                cls = ci * 16 + lane
                ok = cls < N_CLS
                v = plsc.load_gather(deepbuf_v, [delta + cls])
                u = _key(v)
                m = ok & (cls != am_vec) & (u >= t2)

                def dstore(off):
                    offc = jnp.minimum(off, CAND_CAP - 16)
                    plsc.store_compressed(sortk_v.at[pl.ds(offc, 16)], u, mask=m)
                    plsc.store_compressed(
                        sortv_v.at[pl.ds(offc, 16)], flat_base + cls, mask=m
                    )
                    return off + _popcount(m)

                return lax.cond(jnp.any(m), dstore, lambda o: o, off)

            return lax.fori_loop(0, 6, deep_chunk, off)

        n_cand = lax.fori_loop(0, n_deep, deep_row, n_cand)

        nv = CAND_CAP // 16

        def inter_stage(ksz, j):
            jb = j // 16
            s = int(math.log2(jb)) if jb > 0 else 0

            def pair_body(t, _):
                v = ((t >> s) << (s + 1)) | (t & (jb - 1))
                p = v | jb
                ka = sortk_v[pl.ds(v * 16, 16)]
                va = sortv_v[pl.ds(v * 16, 16)]
                kb = sortk_v[pl.ds(p * 16, 16)]
                vb = sortv_v[pl.ds(p * 16, 16)]
                dir_asc = ((v * 16) & ksz) == 0
                lo_before = (ka > kb) | ((ka == kb) & (va < vb))
                swap = lo_before ^ dir_asc
                sortk_v[pl.ds(v * 16, 16)] = jnp.where(swap, kb, ka)
                sortv_v[pl.ds(v * 16, 16)] = jnp.where(swap, vb, va)
                sortk_v[pl.ds(p * 16, 16)] = jnp.where(swap, ka, kb)
                sortv_v[pl.ds(p * 16, 16)] = jnp.where(swap, va, vb)
                return 0

            lax.fori_loop(0, nv // 2, pair_body, 0)

        def intra_stage(ksz, j):
            pidx = lane ^ j

            def vreg_body(v, _):
                ka = sortk_v[pl.ds(v * 16, 16)]
                va = sortv_v[pl.ds(v * 16, 16)]
                kb = _shuf(ka, pidx)
                vb = _shuf(va, pidx)
                am_lower = (lane & j) == 0
                klo = jnp.where(am_lower, ka, kb)
                khi = jnp.where(am_lower, kb, ka)
                vlo = jnp.where(am_lower, va, vb)
                vhi = jnp.where(am_lower, vb, va)
                dir_asc = (((v * 16 + lane) & ksz) == 0)
                lo_before = (klo > khi) | ((klo == khi) & (vlo < vhi))
                swap = lo_before ^ dir_asc
                sortk_v[pl.ds(v * 16, 16)] = jnp.where(swap, kb, ka)
                sortv_v[pl.ds(v * 16, 16)] = jnp.where(swap, vb, va)
                return 0

            lax.fori_loop(0, nv, vreg_body, 0)

        ksz = 2
        while ksz <= CAND_CAP:
            j = ksz // 2
            while j >= 1:
                if j >= 16:
                    inter_stage(ksz, j)
                else:
                    intra_stage(ksz, j)
                j //= 2
            ksz *= 2

        inv91 = jnp.float32(1.0 / N_CLS)

        def out_body(jv, _):
            u = sortk_v[pl.ds(jv * 16, 16)]
            fl = sortv_v[pl.ds(jv * 16, 16)]
            x = _unkey(u)
            scores_v[pl.ds(jv * 16, 16)] = 1.0 / (1.0 + jnp.exp(-x))
            br = (fl.astype(jnp.float32) * inv91).astype(jnp.int32)
            labels_v[pl.ds(jv * 16, 16)] = fl - br * N_CLS
            boxidx_v[pl.ds(jv * 16, 16)] = (b * N_ROWS + br) * 4
            return 0

        lax.fori_loop(0, OUT_W // 16, out_body, 0)

        pltpu.sync_copy(scores_v, scores_out.at[b])
        pltpu.sync_copy(labels_v, labels_out.at[b])

        def bidx_body(jv, _):
            pos = jv * 16 + lane
            base = plsc.load_gather(boxidx_v, [pos >> 2])
            bidx_v[pl.ds(jv * 16, 16)] = base + (pos & 3)
            return 0

        lax.fori_loop(0, OUT_W * 4 // 16, bidx_body, 0)
        pltpu.async_copy(boxes_hbm.at[bidx_v], boxrows_v, sem).wait()

        sc_vec = _shuf(ts_v[pl.ds(0, 16)], (lane & 1) ^ 1)

        def box_body(jv, _):
            pos = jv * 16 + lane
            cl = pos & 3
            v = boxrows_v[pl.ds(jv * 16, 16)]
            vp = plsc.load_gather(boxrows_v, [pos ^ 2])
            xy = jnp.where(cl < 2, v - 0.5 * vp, vp + 0.5 * v)
            boxout_v[pl.ds(jv * 16, 16)] = xy * sc_vec
            return 0

        lax.fori_loop(0, OUT_W * 4 // 16, box_body, 0)
        pltpu.sync_copy(boxout_v, boxes_out.at[b])


def _run_sc(rm, am, m2, logits_flat, boxes_flat, ts_pad):
    mesh = plsc.VectorSubcoreMesh(core_axis_name="c", subcore_axis_name="s")
    f = pl.kernel(
        _sc_body,
        mesh=mesh,
        compiler_params=pltpu.CompilerParams(needs_layout_passes=False),
        out_type=[
            jax.ShapeDtypeStruct((16, OUT_W), jnp.float32),
            jax.ShapeDtypeStruct((16, OUT_W), jnp.int32),
            jax.ShapeDtypeStruct((16, OUT_W * 4), jnp.float32),
        ],
        scratch_types=[
            pltpu.VMEM((N_ROWS,), jnp.float32),
            pltpu.VMEM((N_ROWS,), jnp.int32),
            pltpu.VMEM((N_ROWS,), jnp.float32),
            pltpu.VMEM((GRP_VREGS * 16,), jnp.uint32),
            pltpu.VMEM((CAND_CAP,), jnp.uint32),
            pltpu.VMEM((CAND_CAP,), jnp.int32),
            pltpu.VMEM((DEEP_CAP,), jnp.int32),
            pltpu.VMEM((104,), jnp.float32),
            pltpu.VMEM((OUT_W,), jnp.int32),
            pltpu.VMEM((OUT_W * 4,), jnp.int32),
            pltpu.VMEM((OUT_W * 4,), jnp.float32),
            pltpu.VMEM((OUT_W,), jnp.float32),
            pltpu.VMEM((OUT_W,), jnp.int32),
            pltpu.VMEM((OUT_W * 4,), jnp.float32),
            pltpu.VMEM((16,), jnp.float32),
            pltpu.SemaphoreType.DMA,
        ],
    )
    return f(rm, am, m2, logits_flat, boxes_flat, ts_pad)


def kernel(pred_logits, pred_boxes, target_sizes):
    bs, n, c = pred_logits.shape
    lt = jnp.swapaxes(pred_logits, 1, 2)
    out3 = jax.ShapeDtypeStruct((bs, 1, n), jnp.float32)
    out3i = jax.ShapeDtypeStruct((bs, 1, n), jnp.int32)
    rm, am, m2 = pl.pallas_call(
        _rowstats_body,
        grid=(bs,),
        in_specs=[pl.BlockSpec((1, c, n), lambda b: (b, 0, 0))],
        out_specs=[pl.BlockSpec((1, 1, n), lambda b: (b, 0, 0))] * 3,
        out_shape=[out3, out3i, out3],
    )(lt)
    rm = rm.reshape(bs, n)
    am = am.reshape(bs, n)
    m2 = m2.reshape(bs, n)

    logits_flat = pred_logits.reshape(bs * n * c)
    boxes_flat = pred_boxes.reshape(bs * n * 4)
    ts_pad = jnp.pad(target_sizes, ((0, 0), (0, 14)))

    scores_p, labels_p, boxes_p = _run_sc(
        rm, am, m2, logits_flat, boxes_flat, ts_pad
    )
    scores = scores_p[:, :300]
    labels = labels_p[:, :300]
    boxes = boxes_p.reshape(bs, OUT_W, 4)[:, :300]
    return scores, labels, boxes
